# baseline (speedup 1.0000x reference)
import jax
import jax.numpy as jnp
from jax import lax
from jax.experimental import pallas as pl
from jax.experimental.pallas import tpu as pltpu
from jax.experimental.pallas import tpu_sc as plsc

N = 10000
H = 256
E = 160000
EP = 163840
NP = 10240
NG = 256
T_TREE = 40000
TP_TREE = 40960
PART = 11264
ZROW = 11200
SB = 128
SW = SB + 8


def _mesh():
    return plsc.VectorSubcoreMesh(core_axis_name="c", subcore_axis_name="s")


_NBUF = 4


def _make_gather(t_rows, nch, chunk=64):
    per_w = nch * chunk
    assert nch % _NBUF == 0

    def body(table_hbm, idx_hbm, out_hbm, idx_v, r0, r1, r2, r3, *sems):
        rb = [r0, r1, r2, r3]
        sg, sw = sems[:_NBUF], sems[_NBUF:]
        c = lax.axis_index("c")
        s = lax.axis_index("s")
        w = s * 2 + c
        base = w * per_w
        pltpu.sync_copy(idx_hbm.at[w], idx_v)

        def group(q, carry):
            ds = []
            for b in range(_NBUF):
                j = q * _NBUF + b

                @pl.when(q > 0)
                def _(b=b):
                    pltpu.make_async_copy(
                        rb[b], out_hbm.at[pl.ds(0, chunk)], sw[b]).wait()

                ds.append(pltpu.async_copy(
                    table_hbm.at[idx_v.at[j]], rb[b], sg[b]))
            for b in range(_NBUF):
                j = q * _NBUF + b
                ds[b].wait()
                pltpu.async_copy(
                    rb[b], out_hbm.at[pl.ds(base + j * chunk, chunk)], sw[b])
            return carry

        lax.fori_loop(0, nch // _NBUF, group, 0)
        for b in range(_NBUF):
            pltpu.make_async_copy(
                rb[b], out_hbm.at[pl.ds(0, chunk)], sw[b]).wait()

    return pl.kernel(
        body,
        out_type=jax.ShapeDtypeStruct((32 * per_w, H), jnp.float32),
        mesh=_mesh(),
        scratch_types=(
            [pltpu.VMEM((nch, chunk), jnp.int32)]
            + [pltpu.VMEM((chunk, H), jnp.float32) for _ in range(_NBUF)]
            + [pltpu.SemaphoreType.DMA for _ in range(2 * _NBUF)]
        ),
    )


def _segsum_body(rb_ref, rows_ref, rloc_ref, o_ref, acc):
    i = pl.program_id(0)
    nb = pl.num_programs(0)

    @pl.when(i == 0)
    def _():
        acc[...] = jnp.zeros_like(acc)

    rl = rloc_ref[0, 0, :]
    oh = (lax.broadcasted_iota(jnp.int32, (SW, SB), 0) == rl[None, :]
          ).astype(jnp.float32)
    partial = jnp.dot(oh, rows_ref[...], preferred_element_type=jnp.float32)
    rb = pl.multiple_of(rb_ref[i], 8)
    acc[pl.ds(rb, SW), :] += partial

    @pl.when(i == nb - 1)
    def _():
        o_ref[...] = acc[...]


def _segsum(rows_s, rloc3, rb):
    nb = rows_s.shape[0] // SB
    grid_spec = pltpu.PrefetchScalarGridSpec(
        num_scalar_prefetch=1,
        grid=(nb,),
        in_specs=[pl.BlockSpec((SB, H), lambda i, rb_: (i, 0)),
                  pl.BlockSpec((1, 1, SB), lambda i, rb_: (i, 0, 0))],
        out_specs=pl.BlockSpec((PART, H), lambda i, rb_: (0, 0)),
        scratch_shapes=[pltpu.VMEM((PART, H), jnp.float32)],
    )
    return pl.pallas_call(
        _segsum_body,
        grid_spec=grid_spec,
        out_shape=jax.ShapeDtypeStruct((PART, H), jnp.float32),
    )(rb, rows_s, rloc3)


def _mm_body(a_ref, b_ref, o_ref):
    o_ref[...] = jnp.dot(a_ref[...], b_ref[...],
                         preferred_element_type=jnp.float32)


def _tc_matmul(a, b, blk):
    m, k = a.shape
    n = b.shape[1]
    return pl.pallas_call(
        _mm_body,
        grid=(m // blk,),
        in_specs=[pl.BlockSpec((blk, k), lambda i: (i, 0)),
                  pl.BlockSpec((k, n), lambda i: (0, 0))],
        out_specs=pl.BlockSpec((blk, n), lambda i: (i, 0)),
        out_shape=jax.ShapeDtypeStruct((m, n), jnp.float32),
    )(a, b)


def _mm2_body(a_ref, b_ref, w_ref, o_ref):
    o_ref[...] = jnp.dot(a_ref[...], w_ref[...],
                         preferred_element_type=jnp.float32) + b_ref[...]


def _rank_matmul(part, al_rw, w):
    blk = 2816
    return pl.pallas_call(
        _mm2_body,
        grid=(PART // blk,),
        in_specs=[pl.BlockSpec((blk, H), lambda i: (i, 0)),
                  pl.BlockSpec((blk, H), lambda i: (i, 0)),
                  pl.BlockSpec((H, H), lambda i: (0, 0))],
        out_specs=pl.BlockSpec((blk, H), lambda i: (i, 0)),
        out_shape=jax.ShapeDtypeStruct((PART, H), jnp.float32),
    )(part, al_rw, w)


def _add2_body(a_ref, b_ref, o_ref):
    o_ref[...] = a_ref[...] + b_ref[...]


def _node_add(s, al):
    blk = 2560
    return pl.pallas_call(
        _add2_body,
        grid=(NP // blk,),
        in_specs=[pl.BlockSpec((blk, H), lambda i: (i, 0)),
                  pl.BlockSpec((blk, H), lambda i: (i, 0))],
        out_specs=pl.BlockSpec((blk, H), lambda i: (i, 0)),
        out_shape=jax.ShapeDtypeStruct((NP, H), jnp.float32),
    )(s, al)


_EBLK = 640


def _s2_body(xg_ref, bond_ref, wib_ref, mi_ref, msg_ref):
    mi = xg_ref[...] + jnp.dot(bond_ref[...], wib_ref[...],
                               preferred_element_type=jnp.float32)
    mi_ref[...] = mi
    msg_ref[...] = jnp.maximum(mi, 0.0)


def _edge_init(xg, bond_pad, wibT):
    return pl.pallas_call(
        _s2_body,
        grid=(E // _EBLK,),
        in_specs=[pl.BlockSpec((_EBLK, H), lambda i: (i, 0)),
                  pl.BlockSpec((_EBLK, 8), lambda i: (i, 0)),
                  pl.BlockSpec((8, H), lambda i: (0, 0))],
        out_specs=[pl.BlockSpec((_EBLK, H), lambda i: (i, 0)),
                   pl.BlockSpec((_EBLK, H), lambda i: (i, 0))],
        out_shape=[jax.ShapeDtypeStruct((E, H), jnp.float32),
                   jax.ShapeDtypeStruct((EP, H), jnp.float32)],
    )(xg, bond_pad, wibT)


def _upd_body(msg_ref, mi_ref, g_ref, wh_ref, o_ref):
    mw = jnp.dot(msg_ref[...], wh_ref[...], preferred_element_type=jnp.float32)
    up = jnp.concatenate([mw[1:], mw[:1]], axis=0)
    dn = jnp.concatenate([mw[-1:], mw[:-1]], axis=0)
    rows = lax.broadcasted_iota(jnp.int32, (_EBLK, H), 0)
    sw = jnp.where((rows % 2) == 0, up, dn)
    o_ref[...] = jnp.maximum(mi_ref[...] + g_ref[...] - sw, 0.0)


def _edge_update(msg, mi, g, whT):
    return pl.pallas_call(
        _upd_body,
        grid=(E // _EBLK,),
        in_specs=[pl.BlockSpec((_EBLK, H), lambda i: (i, 0)),
                  pl.BlockSpec((_EBLK, H), lambda i: (i, 0)),
                  pl.BlockSpec((_EBLK, H), lambda i: (i, 0)),
                  pl.BlockSpec((H, H), lambda i: (0, 0))],
        out_specs=pl.BlockSpec((_EBLK, H), lambda i: (i, 0)),
        out_shape=jax.ShapeDtypeStruct((EP, H), jnp.float32),
    )(msg, mi, g, whT)


_FBLK = 400


def _readout_body(x_ref, s_ref, al_ref, gid_ref, wo1_ref, wo2_ref,
                  b_ref, o_ref, acc):
    i = pl.program_id(0)

    @pl.when(i == 0)
    def _():
        acc[...] = jnp.zeros_like(acc)

    m = s_ref[...] + al_ref[...]
    h = jnp.maximum(
        jnp.dot(x_ref[...], wo1_ref[...], preferred_element_type=jnp.float32)
        + jnp.dot(m, wo2_ref[...], preferred_element_type=jnp.float32)
        + b_ref[0:1, :], 0.0)
    gb = gid_ref[0, 0, :]
    oh = (lax.broadcasted_iota(jnp.int32, (NG, _FBLK), 0)
          == gb[None, :]).astype(jnp.float32)
    hcat = jnp.concatenate([h, jnp.ones((_FBLK, 128), jnp.float32)], axis=1)
    acc[...] += jnp.dot(oh, hcat, preferred_element_type=jnp.float32)

    @pl.when(i == (N // _FBLK) - 1)
    def _():
        o_ref[...] = acc[:, :H] / jnp.maximum(acc[:, H:H + 1], 1.0)


def _readout(x_pad, s, al, gid3, wo1T, wo2T, b_pad):
    return pl.pallas_call(
        _readout_body,
        grid=(N // _FBLK,),
        in_specs=[pl.BlockSpec((_FBLK, 128), lambda i: (i, 0)),
                  pl.BlockSpec((_FBLK, H), lambda i: (i, 0)),
                  pl.BlockSpec((_FBLK, H), lambda i: (i, 0)),
                  pl.BlockSpec((1, 1, _FBLK), lambda i: (i, 0, 0)),
                  pl.BlockSpec((128, H), lambda i: (0, 0)),
                  pl.BlockSpec((H, H), lambda i: (0, 0)),
                  pl.BlockSpec((8, H), lambda i: (0, 0))],
        out_specs=pl.BlockSpec((NG, H), lambda i: (0, 0)),
        out_shape=jax.ShapeDtypeStruct((NG, H), jnp.float32),
        scratch_shapes=[pltpu.VMEM((NG, H + 128), jnp.float32)],
    )(x_pad, s, al, gid3, wo1T, wo2T, b_pad)


def _rank_arrays(keys, n_real, n_pad):
    order = jnp.argsort(keys).astype(jnp.int32)
    sk = keys[order]
    newseg = jnp.concatenate(
        [jnp.zeros((1,), jnp.int32), (sk[1:] != sk[:-1]).astype(jnp.int32)])
    r = jnp.cumsum(newseg).astype(jnp.int32)
    rpad = r[-1] + 1 + (jnp.arange(n_pad - n_real, dtype=jnp.int32) & 7)
    r_full = jnp.concatenate([r, rpad])
    perm_full = jnp.concatenate(
        [order, jnp.zeros((n_pad - n_real,), jnp.int32)])
    rb = (r_full.reshape(n_pad // SB, SB)[:, 0] // 8) * 8
    rloc = (r_full - jnp.repeat(rb, SB)).astype(jnp.int32)
    rinv_node = jnp.full((N,), ZROW, jnp.int32).at[sk].set(r)
    rank_inv = jnp.concatenate(
        [rinv_node, jnp.full((NP - N,), ZROW, jnp.int32)])
    nor = jnp.zeros((PART,), jnp.int32).at[r].set(sk)
    return (perm_full, rloc.reshape(n_pad // SB, 1, SB), rb, rank_inv,
            rinv_node, nor)


def kernel(x, bond_feats, tree_alpha, W_i, W_h, W_o, b_o, bonds,
           tree_tgt_nodes, graph_ids):
    src = jnp.stack([bonds[0], bonds[1]], axis=1).reshape(-1).astype(jnp.int32)
    dst = jnp.stack([bonds[1], bonds[0]], axis=1).reshape(-1).astype(jnp.int32)

    perm_e, rloc_e, rb_e, rinv_e, rinv_n, nor_e = _rank_arrays(dst, E, EP)
    perm_t, rloc_t, rb_t, rinv_t, _, _ = _rank_arrays(
        tree_tgt_nodes.astype(jnp.int32), T_TREE, TP_TREE)
    pad_g = jnp.arange(EP - E, dtype=jnp.int32) & 63
    grk = jnp.where(rinv_n[src] == ZROW, PART + src, rinv_n[src])
    grk3 = jnp.concatenate([grk, pad_g]).reshape(32, 80, 64)
    gidx = jnp.concatenate([src, pad_g]).reshape(32, 80, 64)
    perm_e3 = perm_e.reshape(32, 80, 64)
    perm_t3 = perm_t.reshape(32, 20, 64)
    rinv_e3 = rinv_e.reshape(32, 4, 80)
    rinv_t3 = rinv_t.reshape(32, 4, 80)
    nor_e3 = nor_e.reshape(32, 4, 88)

    x_pad = jnp.pad(x, ((0, 0), (0, 128 - 35)))
    wiaT = jnp.pad(W_i[:, :35].T, ((0, 128 - 35), (0, 0)))
    wibT = jnp.pad(W_i[:, 35:].T, ((0, 3), (0, 0)))
    bond_pad = jnp.pad(jnp.repeat(bond_feats, 2, axis=0), ((0, 0), (0, 3)))
    whT = W_h.T
    wo1T = jnp.pad(W_o[:, :35].T, ((0, 128 - 35), (0, 0)))
    wo2T = W_o[:, 35:].T
    b_pad = jnp.broadcast_to(b_o[None, :], (8, H))

    gather_e = _make_gather(EP, 80)
    gather_t = _make_gather(TP_TREE, 20)
    gather_ext = _make_gather(PART + NP, 80)
    gather_n = _make_gather(N, 80)
    gather_rk = _make_gather(PART, 4, 80)
    gather_nor = _make_gather(NP, 4, 88)

    tree_s = gather_t(tree_alpha, perm_t3)
    tpart = _segsum(tree_s, rloc_t, rb_t)
    alpha = gather_rk(tpart, rinv_t3)
    alphaW = _tc_matmul(alpha, whT, 2560)
    alpha_rW = gather_nor(alphaW, nor_e3)
    xw = _tc_matmul(x_pad, wiaT, 2000)
    xg = gather_n(xw, gidx)
    msg_input, msg = _edge_init(xg, bond_pad, wibT)

    for _ in range(5):
        msg_s = gather_e(msg, perm_e3)
        part = _segsum(msg_s, rloc_e, rb_e)
        partW = _rank_matmul(part, alpha_rW, whT)
        tableW = jnp.concatenate([partW, alphaW])
        g = gather_ext(tableW, grk3)
        msg = _edge_update(msg, msg_input, g, whT)

    msg_sf = gather_e(msg, perm_e3)
    segf = gather_rk(_segsum(msg_sf, rloc_e, rb_e), rinv_e3)
    gid3 = graph_ids.astype(jnp.int32).reshape(N // _FBLK, 1, _FBLK)
    return _readout(x_pad, segf, alpha, gid3, wo1T, wo2T, b_pad)


# speedup vs baseline: 1.2169x; 1.2169x over previous
import jax
import jax.numpy as jnp
from jax import lax
from jax.experimental import pallas as pl
from jax.experimental.pallas import tpu as pltpu
from jax.experimental.pallas import tpu_sc as plsc

N = 10000
H = 256
E = 160000
EP = 163840
NP = 10240
NG = 256
T_TREE = 40000
TP_TREE = 40960
PART = 11264
ZROW = 11200
SB = 128
SW = SB + 8


def _mesh():
    return plsc.VectorSubcoreMesh(core_axis_name="c", subcore_axis_name="s")


_NBUF = 4


def _make_gather(t_rows, nch, chunk=64):
    per_w = nch * chunk
    assert nch % _NBUF == 0

    def body(table_hbm, idx_hbm, out_hbm, idx_v, r0, r1, r2, r3, *sems):
        rb = [r0, r1, r2, r3]
        sg, sw = sems[:_NBUF], sems[_NBUF:]
        c = lax.axis_index("c")
        s = lax.axis_index("s")
        w = s * 2 + c
        base = w * per_w
        pltpu.sync_copy(idx_hbm.at[w], idx_v)

        def group(q, carry):
            ds = []
            for b in range(_NBUF):
                j = q * _NBUF + b

                @pl.when(q > 0)
                def _(b=b):
                    pltpu.make_async_copy(
                        rb[b], out_hbm.at[pl.ds(0, chunk)], sw[b]).wait()

                ds.append(pltpu.async_copy(
                    table_hbm.at[idx_v.at[j]], rb[b], sg[b]))
            for b in range(_NBUF):
                j = q * _NBUF + b
                ds[b].wait()
                pltpu.async_copy(
                    rb[b], out_hbm.at[pl.ds(base + j * chunk, chunk)], sw[b])
            return carry

        lax.fori_loop(0, nch // _NBUF, group, 0)
        for b in range(_NBUF):
            pltpu.make_async_copy(
                rb[b], out_hbm.at[pl.ds(0, chunk)], sw[b]).wait()

    return pl.kernel(
        body,
        out_type=jax.ShapeDtypeStruct((32 * per_w, H), jnp.float32),
        mesh=_mesh(),
        scratch_types=(
            [pltpu.VMEM((nch, chunk), jnp.int32)]
            + [pltpu.VMEM((chunk, H), jnp.float32) for _ in range(_NBUF)]
            + [pltpu.SemaphoreType.DMA for _ in range(2 * _NBUF)]
        ),
    )


def _segsum_body(rb_ref, rows_ref, rloc_ref, o_ref, acc):
    i = pl.program_id(0)
    nb = pl.num_programs(0)

    @pl.when(i == 0)
    def _():
        acc[...] = jnp.zeros_like(acc)

    rl = rloc_ref[0, 0, :]
    oh = (lax.broadcasted_iota(jnp.int32, (SW, SB), 0) == rl[None, :]
          ).astype(jnp.float32)
    partial = jnp.dot(oh, rows_ref[...], preferred_element_type=jnp.float32)
    rb = pl.multiple_of(rb_ref[i], 8)
    acc[pl.ds(rb, SW), :] += partial

    @pl.when(i == nb - 1)
    def _():
        o_ref[...] = acc[...]


def _segsum(rows_s, rloc3, rb):
    nb = rows_s.shape[0] // SB
    grid_spec = pltpu.PrefetchScalarGridSpec(
        num_scalar_prefetch=1,
        grid=(nb,),
        in_specs=[pl.BlockSpec((SB, H), lambda i, rb_: (i, 0)),
                  pl.BlockSpec((1, 1, SB), lambda i, rb_: (i, 0, 0))],
        out_specs=pl.BlockSpec((PART, H), lambda i, rb_: (0, 0)),
        scratch_shapes=[pltpu.VMEM((PART, H), jnp.float32)],
    )
    return pl.pallas_call(
        _segsum_body,
        grid_spec=grid_spec,
        out_shape=jax.ShapeDtypeStruct((PART, H), jnp.float32),
    )(rb, rows_s, rloc3)


def _mm_body(a_ref, b_ref, o_ref):
    o_ref[...] = jnp.dot(a_ref[...], b_ref[...],
                         preferred_element_type=jnp.float32)


def _tc_matmul(a, b, blk):
    m, k = a.shape
    n = b.shape[1]
    return pl.pallas_call(
        _mm_body,
        grid=(m // blk,),
        in_specs=[pl.BlockSpec((blk, k), lambda i: (i, 0)),
                  pl.BlockSpec((k, n), lambda i: (0, 0))],
        out_specs=pl.BlockSpec((blk, n), lambda i: (i, 0)),
        out_shape=jax.ShapeDtypeStruct((m, n), jnp.float32),
    )(a, b)


def _mm2_body(a_ref, b_ref, w_ref, o_ref):
    o_ref[...] = jnp.dot(a_ref[...] + b_ref[...], w_ref[...],
                         preferred_element_type=jnp.float32)


def _node_matmul(s, al, w):
    blk = 2560
    return pl.pallas_call(
        _mm2_body,
        grid=(NP // blk,),
        in_specs=[pl.BlockSpec((blk, H), lambda i: (i, 0)),
                  pl.BlockSpec((blk, H), lambda i: (i, 0)),
                  pl.BlockSpec((H, H), lambda i: (0, 0))],
        out_specs=pl.BlockSpec((blk, H), lambda i: (i, 0)),
        out_shape=jax.ShapeDtypeStruct((NP, H), jnp.float32),
    )(s, al, w)


def _rank_matmul(part, al_rw, w):
    blk = 2816
    return pl.pallas_call(
        _mm2_body,
        grid=(PART // blk,),
        in_specs=[pl.BlockSpec((blk, H), lambda i: (i, 0)),
                  pl.BlockSpec((blk, H), lambda i: (i, 0)),
                  pl.BlockSpec((H, H), lambda i: (0, 0))],
        out_specs=pl.BlockSpec((blk, H), lambda i: (i, 0)),
        out_shape=jax.ShapeDtypeStruct((PART, H), jnp.float32),
    )(part, al_rw, w)


def _add2_body(a_ref, b_ref, o_ref):
    o_ref[...] = a_ref[...] + b_ref[...]


def _node_add(s, al):
    blk = 2560
    return pl.pallas_call(
        _add2_body,
        grid=(NP // blk,),
        in_specs=[pl.BlockSpec((blk, H), lambda i: (i, 0)),
                  pl.BlockSpec((blk, H), lambda i: (i, 0))],
        out_specs=pl.BlockSpec((blk, H), lambda i: (i, 0)),
        out_shape=jax.ShapeDtypeStruct((NP, H), jnp.float32),
    )(s, al)


_EBLK = 1600


def _s2_body(xg_ref, bond_ref, wib_ref, mi_ref, msg_ref):
    mi = xg_ref[...] + jnp.dot(bond_ref[...], wib_ref[...],
                               preferred_element_type=jnp.float32)
    mi_ref[...] = mi
    msg_ref[...] = jnp.maximum(mi, 0.0)


def _edge_init(xg, bond_pad, wibT):
    return pl.pallas_call(
        _s2_body,
        grid=(E // _EBLK,),
        in_specs=[pl.BlockSpec((_EBLK, H), lambda i: (i, 0)),
                  pl.BlockSpec((_EBLK, 8), lambda i: (i, 0)),
                  pl.BlockSpec((8, H), lambda i: (0, 0))],
        out_specs=[pl.BlockSpec((_EBLK, H), lambda i: (i, 0)),
                   pl.BlockSpec((_EBLK, H), lambda i: (i, 0))],
        out_shape=[jax.ShapeDtypeStruct((E, H), jnp.float32),
                   jax.ShapeDtypeStruct((EP, H), jnp.float32)],
    )(xg, bond_pad, wibT)


def _upd_body(msg_ref, mi_ref, g_ref, wh_ref, o_ref):
    mw = jnp.dot(msg_ref[...], wh_ref[...], preferred_element_type=jnp.float32)
    up = jnp.concatenate([mw[1:], mw[:1]], axis=0)
    dn = jnp.concatenate([mw[-1:], mw[:-1]], axis=0)
    rows = lax.broadcasted_iota(jnp.int32, (_EBLK, H), 0)
    sw = jnp.where((rows % 2) == 0, up, dn)
    o_ref[...] = jnp.maximum(mi_ref[...] + g_ref[...] - sw, 0.0)


def _edge_update(msg, mi, g, whT):
    return pl.pallas_call(
        _upd_body,
        grid=(E // _EBLK,),
        in_specs=[pl.BlockSpec((_EBLK, H), lambda i: (i, 0)),
                  pl.BlockSpec((_EBLK, H), lambda i: (i, 0)),
                  pl.BlockSpec((_EBLK, H), lambda i: (i, 0)),
                  pl.BlockSpec((H, H), lambda i: (0, 0))],
        out_specs=pl.BlockSpec((_EBLK, H), lambda i: (i, 0)),
        out_shape=jax.ShapeDtypeStruct((EP, H), jnp.float32),
    )(msg, mi, g, whT)


_FBLK = 400


def _readout_body(x_ref, s_ref, al_ref, gid_ref, wo1_ref, wo2_ref,
                  b_ref, o_ref, acc):
    i = pl.program_id(0)

    @pl.when(i == 0)
    def _():
        acc[...] = jnp.zeros_like(acc)

    m = s_ref[...] + al_ref[...]
    h = jnp.maximum(
        jnp.dot(x_ref[...], wo1_ref[...], preferred_element_type=jnp.float32)
        + jnp.dot(m, wo2_ref[...], preferred_element_type=jnp.float32)
        + b_ref[0:1, :], 0.0)
    gb = gid_ref[0, 0, :]
    oh = (lax.broadcasted_iota(jnp.int32, (NG, _FBLK), 0)
          == gb[None, :]).astype(jnp.float32)
    hcat = jnp.concatenate([h, jnp.ones((_FBLK, 128), jnp.float32)], axis=1)
    acc[...] += jnp.dot(oh, hcat, preferred_element_type=jnp.float32)

    @pl.when(i == (N // _FBLK) - 1)
    def _():
        o_ref[...] = acc[:, :H] / jnp.maximum(acc[:, H:H + 1], 1.0)


def _readout(x_pad, s, al, gid3, wo1T, wo2T, b_pad):
    return pl.pallas_call(
        _readout_body,
        grid=(N // _FBLK,),
        in_specs=[pl.BlockSpec((_FBLK, 128), lambda i: (i, 0)),
                  pl.BlockSpec((_FBLK, H), lambda i: (i, 0)),
                  pl.BlockSpec((_FBLK, H), lambda i: (i, 0)),
                  pl.BlockSpec((1, 1, _FBLK), lambda i: (i, 0, 0)),
                  pl.BlockSpec((128, H), lambda i: (0, 0)),
                  pl.BlockSpec((H, H), lambda i: (0, 0)),
                  pl.BlockSpec((8, H), lambda i: (0, 0))],
        out_specs=pl.BlockSpec((NG, H), lambda i: (0, 0)),
        out_shape=jax.ShapeDtypeStruct((NG, H), jnp.float32),
        scratch_shapes=[pltpu.VMEM((NG, H + 128), jnp.float32)],
    )(x_pad, s, al, gid3, wo1T, wo2T, b_pad)


def _rank_arrays(keys, n_real, n_pad):
    order = jnp.argsort(keys).astype(jnp.int32)
    sk = keys[order]
    newseg = jnp.concatenate(
        [jnp.zeros((1,), jnp.int32), (sk[1:] != sk[:-1]).astype(jnp.int32)])
    r = jnp.cumsum(newseg).astype(jnp.int32)
    rpad = r[-1] + 1 + (jnp.arange(n_pad - n_real, dtype=jnp.int32) & 7)
    r_full = jnp.concatenate([r, rpad])
    perm_full = jnp.concatenate(
        [order, jnp.zeros((n_pad - n_real,), jnp.int32)])
    rb = (r_full.reshape(n_pad // SB, SB)[:, 0] // 8) * 8
    rloc = (r_full - jnp.repeat(rb, SB)).astype(jnp.int32)
    rinv_node = jnp.full((N,), ZROW, jnp.int32).at[sk].set(r)
    rank_inv = jnp.concatenate(
        [rinv_node, jnp.full((NP - N,), ZROW, jnp.int32)])
    nor = jnp.zeros((PART,), jnp.int32).at[r].set(sk)
    return (perm_full, rloc.reshape(n_pad // SB, 1, SB), rb, rank_inv,
            rinv_node, nor)


def kernel(x, bond_feats, tree_alpha, W_i, W_h, W_o, b_o, bonds,
           tree_tgt_nodes, graph_ids):
    src = jnp.stack([bonds[0], bonds[1]], axis=1).reshape(-1).astype(jnp.int32)
    dst = jnp.stack([bonds[1], bonds[0]], axis=1).reshape(-1).astype(jnp.int32)

    perm_e, rloc_e, rb_e, rinv_e, rinv_n, nor_e = _rank_arrays(dst, E, EP)
    perm_t, rloc_t, rb_t, rinv_t, _, _ = _rank_arrays(
        tree_tgt_nodes.astype(jnp.int32), T_TREE, TP_TREE)
    pad_g = jnp.arange(EP - E, dtype=jnp.int32) & 63
    grk = jnp.where(rinv_n[src] == ZROW, PART + src, rinv_n[src])
    grk3 = jnp.concatenate([grk, pad_g]).reshape(32, 80, 64)
    gidx = jnp.concatenate([src, pad_g]).reshape(32, 80, 64)
    perm_e3 = perm_e.reshape(32, 80, 64)
    perm_t3 = perm_t.reshape(32, 20, 64)
    rinv_e3 = rinv_e.reshape(32, 4, 80)
    rinv_t3 = rinv_t.reshape(32, 4, 80)
    nor_e3 = nor_e.reshape(32, 4, 88)

    x_pad = jnp.pad(x, ((0, 0), (0, 128 - 35)))
    wiaT = jnp.pad(W_i[:, :35].T, ((0, 128 - 35), (0, 0)))
    wibT = jnp.pad(W_i[:, 35:].T, ((0, 3), (0, 0)))
    bond_pad = jnp.pad(jnp.repeat(bond_feats, 2, axis=0), ((0, 0), (0, 3)))
    whT = W_h.T
    wo1T = jnp.pad(W_o[:, :35].T, ((0, 128 - 35), (0, 0)))
    wo2T = W_o[:, 35:].T
    b_pad = jnp.broadcast_to(b_o[None, :], (8, H))

    gather_e = _make_gather(EP, 80)
    gather_t = _make_gather(TP_TREE, 20)
    gather_np = _make_gather(NP, 80)
    gather_n = _make_gather(N, 80)
    gather_rk = _make_gather(PART, 4, 80)
    tree_s = gather_t(tree_alpha, perm_t3)
    tpart = _segsum(tree_s, rloc_t, rb_t)
    alpha = gather_rk(tpart, rinv_t3)
    xw = _tc_matmul(x_pad, wiaT, 2000)
    xg = gather_n(xw, gidx)
    msg_input, msg = _edge_init(xg, bond_pad, wibT)

    for _ in range(5):
        msg_s = gather_e(msg, perm_e3)
        part = _segsum(msg_s, rloc_e, rb_e)
        seg = gather_rk(part, rinv_e3)
        nodeW = _node_matmul(seg, alpha, whT)
        g = gather_np(nodeW, gidx)
        msg = _edge_update(msg, msg_input, g, whT)

    msg_sf = gather_e(msg, perm_e3)
    segf = gather_rk(_segsum(msg_sf, rloc_e, rb_e), rinv_e3)
    gid3 = graph_ids.astype(jnp.int32).reshape(N // _FBLK, 1, _FBLK)
    return _readout(x_pad, segf, alpha, gid3, wo1T, wo2T, b_pad)
